# baseline (speedup 1.0000x reference)
import functools
import math

import jax
import jax.numpy as jnp
from jax import lax
from jax.experimental import pallas as pl
from jax.experimental.pallas import tpu as pltpu
from jax.experimental.pallas import tpu_sc as plsc

KERNEL_SIZE_FACTOR = 0.1
SIGMA = 3.0


def _kernel_consts(H):
    ks = int(H * KERNEL_SIZE_FACTOR)
    if ks % 2 == 0:
        ks += 1
    half = ks // 2
    c = (ks - 1) / 2.0
    z = sum(math.exp(-((i - c) ** 2) / (2.0 * SIGMA**2)) for i in range(ks))
    return ks, half, 1.0 / z


_NC = 2
_NS = 16
_L = 16


def _sc_scatter_counts(pts_t, B, P, H, W):
    MAP = H * W
    BPC = B // _NC
    TPB = _NS // BPC
    ROWS = H // TPB
    SLICE = ROWS * W

    mesh = plsc.VectorSubcoreMesh(
        core_axis_name="c", subcore_axis_name="s"
    )

    @functools.partial(
        pl.kernel,
        out_type=jax.ShapeDtypeStruct((B * H, W), jnp.float32),
        mesh=mesh,
        scratch_types=[
            pltpu.VMEM((P,), jnp.float32),
            pltpu.VMEM((P,), jnp.float32),
            pltpu.VMEM((ROWS, W), jnp.float32),
        ],
        compiler_params=pltpu.CompilerParams(needs_layout_passes=False),
    )
    def sc_scatter(pts_hbm, out_hbm, xv, yv, cslice):
        c = lax.axis_index("c")
        s = lax.axis_index("s")
        b = c * BPC + s // TPB
        r0 = (s % TPB) * ROWS

        pltpu.sync_copy(pts_hbm.at[0, b], xv)
        pltpu.sync_copy(pts_hbm.at[1, b], yv)

        zeros = jnp.zeros((_L,), jnp.float32)

        def zbody(i, carry):
            for k in range(W // _L):
                cslice[i, pl.ds(k * _L, _L)] = zeros
            return carry

        lax.fori_loop(0, ROWS, zbody, 0)

        ones = jnp.ones((_L,), jnp.float32)

        def sbody(i, carry):
            x16 = xv[pl.ds(i * _L, _L)]
            y16 = yv[pl.ds(i * _L, _L)]
            xi = (x16 * W).astype(jnp.int32)
            yi = (y16 * H).astype(jnp.int32) - r0
            mask = (yi >= 0) & (yi < ROWS)
            yis = jnp.where(mask, yi, 0)
            plsc.addupdate_scatter(cslice, [yis, xi], ones, mask=mask)
            return carry

        lax.fori_loop(0, P // _L, sbody, 0)

        pltpu.sync_copy(cslice, out_hbm.at[pl.ds(b * H + r0, ROWS), :])

    return sc_scatter(pts_t)


def _t_build_body(t_ref, *, H, half, inv_z):
    r = lax.broadcasted_iota(jnp.int32, (H, H), 0).astype(jnp.float32)
    cc = lax.broadcasted_iota(jnp.int32, (H, H), 1).astype(jnp.float32)
    d = r - cc
    inv_two_sigma2 = -1.0 / (2.0 * SIGMA * SIGMA)
    t_ref[...] = jnp.where(
        jnp.abs(d) <= half,
        jnp.exp(d * d * inv_two_sigma2) * inv_z,
        0.0,
    )


def _tc_build_t(H, half, inv_z):
    body = functools.partial(_t_build_body, H=H, half=half, inv_z=inv_z)
    return pl.pallas_call(
        body,
        out_shape=jax.ShapeDtypeStruct((H, H), jnp.float32),
    )()


def _tc_blur_body(t_ref, cnt_hbm, o_ref, mbuf, sems, *, B, H, W):
    b = pl.program_id(0)
    slot = lax.rem(b, 2)
    nslot = 1 - slot

    @pl.when(b == 0)
    def _():
        pltpu.make_async_copy(
            cnt_hbm.at[pl.ds(0, H), :], mbuf.at[0], sems.at[0]
        ).start()

    @pl.when(b + 1 < B)
    def _():
        pltpu.make_async_copy(
            cnt_hbm.at[pl.ds((b + 1) * H, H), :], mbuf.at[nslot],
            sems.at[nslot],
        ).start()

    pltpu.make_async_copy(
        cnt_hbm.at[pl.ds(b * H, H), :], mbuf.at[slot], sems.at[slot]
    ).wait()
    T = t_ref[...]
    M = mbuf[slot]
    A = lax.dot_general(
        T, M, (((1,), (0,)), ((), ())),
        preferred_element_type=jnp.float32,
    )
    o_ref[0] = lax.dot_general(
        A, T, (((1,), (0,)), ((), ())),
        preferred_element_type=jnp.float32,
    )


def _tc_blur(t_mat, counts_flat, B, H, W):
    body = functools.partial(_tc_blur_body, B=B, H=H, W=W)
    return pl.pallas_call(
        body,
        grid=(B,),
        in_specs=[
            pl.BlockSpec((H, H), lambda b: (0, 0)),
            pl.BlockSpec(memory_space=pl.ANY),
        ],
        out_specs=pl.BlockSpec((1, H, W), lambda b: (b, 0, 0)),
        out_shape=jax.ShapeDtypeStruct((B, H, W), jnp.float32),
        scratch_shapes=[
            pltpu.VMEM((2, H, W), jnp.float32),
            pltpu.SemaphoreType.DMA((2,)),
        ],
    )(t_mat, counts_flat)


def kernel(feature_map, points):
    B, C, H, W = feature_map.shape
    P = points.shape[1]
    ks, half, inv_z = _kernel_consts(min(H, W))

    pts_t = jnp.transpose(points, (2, 0, 1))

    t_mat = _tc_build_t(H, half, inv_z)
    counts_flat = _sc_scatter_counts(pts_t, B, P, H, W)
    return _tc_blur(t_mat, counts_flat, B, H, W)
